# Initial kernel scaffold; baseline (speedup 1.0000x reference)
import jax
import jax.numpy as jnp
from jax.experimental import pallas as pl


def kernel(encoder_out, align_phone, text_phone, pitch, beats, singer_vec, W_pitch, b_pitch, W_pos, b_pos, emb_beats, emb_singer, W_out, b_out):
    raise NotImplementedError("write your pallas kernel here")


# speedup vs baseline: 15.1824x; 15.1824x over previous
import functools

import numpy as np
import jax
import jax.numpy as jnp
from jax import lax
from jax.experimental import pallas as pl
from jax.experimental.pallas import tpu as pltpu
from jax.experimental.pallas import tpu_sc as plsc

_HIGH = lax.Precision.HIGHEST


def _dgT(x, w):
    return lax.dot_general(x, w, (((1,), (1,)), ((), ())),
                           precision=_HIGH, preferred_element_type=jnp.float32)


def _pe_const(t, d):
    pos = np.arange(t)[:, None].astype(np.float32)
    div = np.exp(np.arange(0, d, 2).astype(np.float32) * (-np.log(10000.0) / d))
    pe = np.zeros((t, d), dtype=np.float32)
    pe[:, 0::2] = np.sin(pos * div)
    pe[:, 1::2] = np.cos(pos * div)
    return jnp.asarray(pe)


def _sc_scan_gather(align, text, enc):
    B, T = align.shape
    D = enc.shape[2]
    assert B == 16
    CH = 64
    half_t = T // 2
    alignT = jnp.swapaxes(align, 0, 1).reshape(-1)
    mesh = plsc.VectorSubcoreMesh(core_axis_name="c", subcore_axis_name="s")

    @functools.partial(
        pl.kernel,
        mesh=mesh,
        compiler_params=pltpu.CompilerParams(needs_layout_passes=False),
        out_type=jax.ShapeDtypeStruct((B, T, D), jnp.float32),
        scratch_types=[
            pltpu.VMEM((T * 16,), jnp.int32),
            pltpu.VMEM((B, T), jnp.int32),
            pltpu.VMEM((T * 16,), jnp.int32),
            pltpu.VMEM((CH,), jnp.int32),
            pltpu.VMEM((CH, D), jnp.float32),
            pltpu.SemaphoreType.DMA,
        ],
    )
    def k(alignT_hbm, text_hbm, enc_hbm, out_hbm,
          alignT_v, text_v, indsT_v, idx_v, rows_v, sem):
        cid = lax.axis_index("c")
        sid = lax.axis_index("s")
        wid = sid * 2 + cid
        b = wid // 2
        half = wid % 2
        pltpu.sync_copy(alignT_hbm, alignT_v)
        pltpu.sync_copy(text_hbm, text_v)
        lane = lax.iota(jnp.int32, 16)
        zero16 = jnp.zeros((16,), jnp.int32)
        indsT_v[pl.ds(0, 16)] = zero16

        def scan_body(t, ind_vec):
            a = alignT_v[pl.ds(t * 16, 16)]
            tx = plsc.load_gather(text_v, [lane, ind_vec])
            ind_vec = ind_vec + jnp.where(a != tx, 1, 0).astype(jnp.int32)
            indsT_v[pl.ds(t * 16, 16)] = ind_vec
            return ind_vec

        lax.fori_loop(1, T, scan_body, zero16)

        base = half * half_t

        def chunk_body(ch, _):
            off = base + ch * CH
            for j in range(CH // 16):
                idx_v[pl.ds(j * 16, 16)] = plsc.load_gather(
                    indsT_v, [(off + j * 16 + lane) * 16 + b])
            pltpu.async_copy(enc_hbm.at[b].at[idx_v], rows_v, sem).wait()
            pltpu.sync_copy(rows_v, out_hbm.at[b].at[pl.ds(off, CH)])
            return 0

        lax.fori_loop(0, half_t // CH, chunk_body, 0)

    return k(alignT, text, enc)


def _tc_precompute(W_pos, W_out, small, b_out_row, emb_singer, pe):
    D = W_pos.shape[0]
    T = pe.shape[0]

    def body(wpos_ref, wout_ref, small_ref, bout_ref, sing_ref, pe_ref,
             a2_ref, pe2_ref, s2_ref, misc_ref):
        W1 = wout_ref[:, :D]
        W2 = wout_ref[:, D:]
        Wp = wpos_ref[...]
        a2_ref[...] = W1 + jnp.dot(W1, Wp, precision=_HIGH,
                                   preferred_element_type=jnp.float32)
        m8 = _dgT(small_ref[...], W1)
        misc_ref[...] = m8
        pe2_ref[...] = _dgT(_dgT(pe_ref[...], Wp), W1) + m8[1:2, :] + bout_ref[...]
        s2_ref[...] = _dgT(sing_ref[...], W2)

    return pl.pallas_call(
        body,
        out_shape=(
            jax.ShapeDtypeStruct((D, D), jnp.float32),
            jax.ShapeDtypeStruct((T, D), jnp.float32),
            jax.ShapeDtypeStruct((emb_singer.shape[0], D), jnp.float32),
            jax.ShapeDtypeStruct((8, D), jnp.float32),
        ),
    )(W_pos, W_out, small, b_out_row, emb_singer, pe)


def _tc_main(G, S, PE2c, A2, S2, misc):
    B, T, D = G.shape
    TILE = 512
    NS = S2.shape[0]

    def body(g_ref, s_ref, pe2_ref, a2_ref, s2_ref, misc_ref, o_ref):
        g = g_ref[0]
        acc = _dgT(g, a2_ref[...])
        acc = acc + pe2_ref[...]
        sc = s_ref[0]
        acc = acc + sc[:, 0:1] * misc_ref[0:1, :]
        acc = acc + sc[:, 1:2] * misc_ref[2:3, :]
        oneh = (sc[:, 2:3].astype(jnp.int32)
                == lax.broadcasted_iota(jnp.int32, (1, NS), 1))
        acc = acc + jnp.dot(oneh.astype(jnp.float32), s2_ref[...],
                            precision=_HIGH, preferred_element_type=jnp.float32)
        o_ref[0] = jnp.where(acc >= 0, acc, 0.01 * acc)

    return pl.pallas_call(
        body,
        grid=(B, T // TILE),
        in_specs=[
            pl.BlockSpec((1, TILE, D), lambda b, t: (b, t, 0)),
            pl.BlockSpec((1, TILE, 4), lambda b, t: (b, t, 0)),
            pl.BlockSpec((TILE, D), lambda b, t: (t, 0)),
            pl.BlockSpec((D, D), lambda b, t: (0, 0)),
            pl.BlockSpec((NS, D), lambda b, t: (0, 0)),
            pl.BlockSpec((8, D), lambda b, t: (0, 0)),
        ],
        out_specs=pl.BlockSpec((1, TILE, D), lambda b, t: (b, t, 0)),
        out_shape=jax.ShapeDtypeStruct((B, T, D), jnp.float32),
    )(G, S, PE2c, A2, S2, misc)


def kernel(encoder_out, align_phone, text_phone, pitch, beats, singer_vec,
           W_pitch, b_pitch, W_pos, b_pos, emb_beats, emb_singer, W_out, b_out):
    B, T, D = encoder_out.shape
    pe = _pe_const(T, D)

    G = _sc_scan_gather(align_phone.astype(jnp.int32),
                        text_phone.astype(jnp.int32), encoder_out)

    small = jnp.zeros((8, D), jnp.float32)
    small = small.at[0].set(W_pitch[:, 0])
    small = small.at[1].set(b_pitch + b_pos + emb_beats[0])
    small = small.at[2].set(emb_beats[1] - emb_beats[0])
    A2, PE2c, S2, misc = _tc_precompute(W_pos, W_out, small,
                                        b_out.reshape(1, D), emb_singer, pe)

    S = jnp.concatenate(
        [pitch, beats.astype(jnp.float32), singer_vec.astype(jnp.float32),
         jnp.zeros_like(pitch)], axis=2)

    return _tc_main(G, S, PE2c, A2, S2, misc)
